# Initial kernel scaffold; baseline (speedup 1.0000x reference)
import jax
import jax.numpy as jnp
from jax.experimental import pallas as pl


def kernel(x, segment, tok_table, seg_table, pos_table, gamma, beta):
    raise NotImplementedError("write your pallas kernel here")


# speedup vs baseline: 3.6744x; 3.6744x over previous
import functools

import jax
import jax.numpy as jnp
from jax import lax
from jax.experimental import pallas as pl
from jax.experimental.pallas import tpu as pltpu
from jax.experimental.pallas import tpu_sc as plsc

B, T, DIM = 1024, 200, 64
N_TOK = B * T


_NW = 32
_CHUNK = 128
_PER_W = N_TOK // _NW
_NCHUNK = _PER_W // _CHUNK


def _sc_gather(tok_table, idx_flat):
    mesh = plsc.VectorSubcoreMesh(core_axis_name="c", subcore_axis_name="s")

    @functools.partial(
        pl.kernel,
        out_type=jax.ShapeDtypeStruct((N_TOK, DIM), jnp.float32),
        mesh=mesh,
        scratch_types=[
            pltpu.VMEM((_CHUNK,), jnp.int32),
            pltpu.VMEM((_CHUNK, DIM), jnp.float32),
            pltpu.SemaphoreType.DMA,
        ],
        compiler_params=pltpu.CompilerParams(use_tc_tiling_on_sc=False),
    )
    def k(table_hbm, idx_hbm, out_hbm, idx_v, rows_v, sem):
        wid = lax.axis_index("s") * 2 + lax.axis_index("c")
        base = wid * _PER_W

        def body(i, _):
            off = base + i * _CHUNK
            pltpu.sync_copy(idx_hbm.at[pl.ds(off, _CHUNK)], idx_v)
            pltpu.async_copy(table_hbm.at[idx_v], rows_v, sem).wait()
            pltpu.sync_copy(rows_v, out_hbm.at[pl.ds(off, _CHUNK)])
            return ()

        lax.fori_loop(0, _NCHUNK, body, ())

    return k(tok_table, idx_flat)


_BBLK = 64


def _tc_ln_body(emb_ref, seg_ref, segtab_ref, postab_ref, gamma_ref,
                beta_ref, out_ref):
    emb = emb_ref[...]
    seg = seg_ref[...].astype(jnp.float32)
    s0 = segtab_ref[0, :]
    s1 = segtab_ref[1, :]
    seg_emb = s0[None, None, :] + seg[:, :, None] * (s1 - s0)[None, None, :]
    x = emb + seg_emb + postab_ref[...][None, :, :]
    mean = jnp.mean(x, axis=-1, keepdims=True)
    xc = x - mean
    var = jnp.mean(xc * xc, axis=-1, keepdims=True)
    y = xc * lax.rsqrt(var + 1e-5)
    out_ref[...] = y * gamma_ref[...][None, None, :] + beta_ref[...][None, None, :]


def _tc_ln(emb, segment, seg_table, pos_slice, gamma, beta):
    grid = (B // _BBLK,)
    return pl.pallas_call(
        _tc_ln_body,
        grid=grid,
        in_specs=[
            pl.BlockSpec((_BBLK, T, DIM), lambda i: (i, 0, 0)),
            pl.BlockSpec((_BBLK, T), lambda i: (i, 0)),
            pl.BlockSpec((2, DIM), lambda i: (0, 0)),
            pl.BlockSpec((T, DIM), lambda i: (0, 0)),
            pl.BlockSpec((DIM,), lambda i: (0,)),
            pl.BlockSpec((DIM,), lambda i: (0,)),
        ],
        out_specs=pl.BlockSpec((_BBLK, T, DIM), lambda i: (i, 0, 0)),
        out_shape=jax.ShapeDtypeStruct((B, T, DIM), jnp.float32),
    )(emb, segment, seg_table, pos_slice, gamma, beta)


def kernel(x, segment, tok_table, seg_table, pos_table, gamma, beta):
    idx_flat = x.reshape(N_TOK).astype(jnp.int32)
    emb = _sc_gather(tok_table, idx_flat)
    emb = emb.reshape(B, T, DIM)
    return _tc_ln(emb, segment, seg_table, pos_table[:T], gamma, beta)
